# Initial kernel scaffold; baseline (speedup 1.0000x reference)
import jax
import jax.numpy as jnp
from jax.experimental import pallas as pl


def kernel(x, edge_index, edge_attr, batch, atom_embed, bond_embed, W1, b1, W2, b2, eps, gamma, beta):
    raise NotImplementedError("write your pallas kernel here")


# speedup vs baseline: 4.3593x; 4.3593x over previous
import functools

import jax
import jax.numpy as jnp
from jax import lax
from jax.experimental import pallas as pl
from jax.experimental.pallas import tpu as pltpu
from jax.experimental.pallas import tpu_sc as plsc

N = 10000
E = 320000
D = 128
B = 400
NP = 10240
R = 512
G = NP // R
NB = 8

NC, NS = 2, 16
NW = NC * NS
EPT = E // NW
CH = 80
NCH = EPT // CH
ZR = NP // NS


def _edge_agg_kernel(src_hbm, bnd_hbm, dst_hbm, t_hbm, out_hbm,
                     src_v, b_v, dst_v, idx_v, rows_v, acc_sh, sem):
    c = lax.axis_index("c")
    s = lax.axis_index("s")
    tid = s * NC + c

    def zrow(i, carry):
        for j in range(D // 16):
            rows_v[i, pl.ds(j * 16, 16)] = jnp.zeros((16,), jnp.float32)
        return carry
    lax.fori_loop(0, CH, zrow, 0)

    def zacc(k, carry):
        pltpu.sync_copy(rows_v, acc_sh.at[pl.ds(s * ZR + k * CH, CH)])
        return carry
    lax.fori_loop(0, ZR // CH, zacc, 0)
    plsc.subcore_barrier()

    base = tid * EPT

    def chunk(i, carry):
        off = base + i * CH
        pltpu.sync_copy(src_hbm.at[pl.ds(off, CH)], src_v)
        pltpu.sync_copy(bnd_hbm.at[pl.ds(off, CH)], b_v)
        pltpu.sync_copy(dst_hbm.at[pl.ds(off, CH)], dst_v)
        for j in range(CH // 16):
            sl = pl.ds(j * 16, 16)
            idx_v[sl] = b_v[sl] * NP + src_v[sl]
        pltpu.async_copy(t_hbm.at[idx_v], rows_v, sem).wait()
        pltpu.sync_copy(rows_v, acc_sh.at[dst_v], add=True)
        return carry
    lax.fori_loop(0, NCH, chunk, 0)
    plsc.subcore_barrier()

    def cpout(k, carry):
        r = s * ZR + k * CH
        pltpu.sync_copy(acc_sh.at[pl.ds(r, CH)], rows_v)
        pltpu.sync_copy(rows_v, out_hbm.at[pl.ds(c * NP + r, CH)])
        return carry
    lax.fori_loop(0, ZR // CH, cpout, 0)


_edge_agg = functools.partial(
    pl.kernel,
    mesh=plsc.VectorSubcoreMesh(core_axis_name="c", subcore_axis_name="s"),
    out_type=jax.ShapeDtypeStruct((NC * NP, D), jnp.float32),
    scratch_types=[
        pltpu.VMEM((CH,), jnp.int32),
        pltpu.VMEM((CH,), jnp.int32),
        pltpu.VMEM((CH,), jnp.int32),
        pltpu.VMEM((CH,), jnp.int32),
        pltpu.VMEM((CH, D), jnp.float32),
        pltpu.VMEM_SHARED((NP, D), jnp.float32),
        pltpu.SemaphoreType.DMA,
    ],
)(_edge_agg_kernel)


def _embed_body(x0_ref, ae_ref, bond_ref, h_ref, t_ref):
    xrow = x0_ref[0]
    iota = lax.broadcasted_iota(jnp.int32, (119, R), 0).astype(jnp.float32)
    onehot_t = (jnp.broadcast_to(xrow, (119, R)) == iota).astype(jnp.float32)
    h = lax.dot_general(onehot_t, ae_ref[...], (((0,), (0,)), ((), ())),
                        precision=lax.Precision.HIGHEST,
                        preferred_element_type=jnp.float32)
    h_ref[...] = h
    for b in range(NB):
        t_ref[b] = jnp.maximum(h + bond_ref[b:b + 1, :], 0.0)


_embed = pl.pallas_call(
    _embed_body,
    grid=(G,),
    in_specs=[
        pl.BlockSpec((1, 1, R), lambda i: (i, 0, 0)),
        pl.BlockSpec((119, D), lambda i: (0, 0)),
        pl.BlockSpec((NB, D), lambda i: (0, 0)),
    ],
    out_specs=[
        pl.BlockSpec((R, D), lambda i: (i, 0)),
        pl.BlockSpec((NB, R, D), lambda i: (0, i, 0)),
    ],
    out_shape=[
        jax.ShapeDtypeStruct((NP, D), jnp.float32),
        jax.ShapeDtypeStruct((NB, NP, D), jnp.float32),
    ],
)


def _mlp_body(h_ref, a0_ref, a1_ref, eps_ref, w1_ref, b1_ref, w2_ref, b2_ref,
              y_ref, st_ref):
    i = pl.program_id(0)
    u = eps_ref[...] * h_ref[...] + a0_ref[...] + a1_ref[...]
    t = jnp.maximum(jnp.dot(u, w1_ref[...],
                            preferred_element_type=jnp.float32) + b1_ref[...],
                    0.0)
    y = jnp.dot(t, w2_ref[...],
                preferred_element_type=jnp.float32) + b2_ref[...]
    y_ref[...] = y

    @pl.when(i == 0)
    def _():
        st_ref[...] = jnp.zeros_like(st_ref)

    row = i * R + lax.broadcasted_iota(jnp.int32, (R, D), 0)
    ym = jnp.where(row < N, y, 0.0)
    s1 = jnp.sum(ym, axis=0, keepdims=True)
    st_ref[...] += jnp.concatenate([s1, jnp.zeros((7, D), jnp.float32)], 0)


_mlp = pl.pallas_call(
    _mlp_body,
    grid=(G,),
    in_specs=[
        pl.BlockSpec((R, D), lambda i: (i, 0)),
        pl.BlockSpec((R, D), lambda i: (i, 0)),
        pl.BlockSpec((R, D), lambda i: (i + G, 0)),
        pl.BlockSpec((1, D), lambda i: (0, 0)),
        pl.BlockSpec((D, 2 * D), lambda i: (0, 0)),
        pl.BlockSpec((1, 2 * D), lambda i: (0, 0)),
        pl.BlockSpec((2 * D, D), lambda i: (0, 0)),
        pl.BlockSpec((1, D), lambda i: (0, 0)),
    ],
    out_specs=[
        pl.BlockSpec((R, D), lambda i: (i, 0)),
        pl.BlockSpec((8, D), lambda i: (0, 0)),
    ],
    out_shape=[
        jax.ShapeDtypeStruct((NP, D), jnp.float32),
        jax.ShapeDtypeStruct((8, D), jnp.float32),
    ],
)


def _var_body(stm_ref, y_ref, stv_ref):
    i = pl.program_id(0)

    @pl.when(i == 0)
    def _():
        stv_ref[...] = jnp.zeros_like(stv_ref)

    mean = stm_ref[0:1, :] * (1.0 / N)
    row = i * R + lax.broadcasted_iota(jnp.int32, (R, D), 0)
    yc = jnp.where(row < N, y_ref[...] - mean, 0.0)
    s2 = jnp.sum(yc * yc, axis=0, keepdims=True)
    stv_ref[...] += jnp.concatenate([s2, jnp.zeros((7, D), jnp.float32)], 0)


_var = pl.pallas_call(
    _var_body,
    grid=(G,),
    in_specs=[
        pl.BlockSpec((8, D), lambda i: (0, 0)),
        pl.BlockSpec((R, D), lambda i: (i, 0)),
    ],
    out_specs=pl.BlockSpec((8, D), lambda i: (0, 0)),
    out_shape=jax.ShapeDtypeStruct((8, D), jnp.float32),
)


def _bn_core(stm_ref, stv_ref, y_ref, gamma_ref, beta_ref):
    mean = stm_ref[0:1, :] * (1.0 / N)
    var = stv_ref[0:1, :] * (1.0 / N)
    inv = gamma_ref[...] / jnp.sqrt(var + 1e-5)
    return (y_ref[...] - mean) * inv + beta_ref[...]


def _finalize_body(stm_ref, stv_ref, y_ref, gamma_ref, beta_ref, bond_ref,
                   h_ref, t_ref):
    h = jnp.maximum(_bn_core(stm_ref, stv_ref, y_ref, gamma_ref, beta_ref), 0.0)
    h_ref[...] = h
    for b in range(NB):
        t_ref[b] = jnp.maximum(h + bond_ref[b:b + 1, :], 0.0)


_finalize = pl.pallas_call(
    _finalize_body,
    grid=(G,),
    in_specs=[
        pl.BlockSpec((8, D), lambda i: (0, 0)),
        pl.BlockSpec((8, D), lambda i: (0, 0)),
        pl.BlockSpec((R, D), lambda i: (i, 0)),
        pl.BlockSpec((1, D), lambda i: (0, 0)),
        pl.BlockSpec((1, D), lambda i: (0, 0)),
        pl.BlockSpec((NB, D), lambda i: (0, 0)),
    ],
    out_specs=[
        pl.BlockSpec((R, D), lambda i: (i, 0)),
        pl.BlockSpec((NB, R, D), lambda i: (0, i, 0)),
    ],
    out_shape=[
        jax.ShapeDtypeStruct((NP, D), jnp.float32),
        jax.ShapeDtypeStruct((NB, NP, D), jnp.float32),
    ],
)


def _finalize_last_body(stm_ref, stv_ref, y_ref, gamma_ref, beta_ref, h_ref):
    h_ref[...] = _bn_core(stm_ref, stv_ref, y_ref, gamma_ref, beta_ref)


_finalize_last = pl.pallas_call(
    _finalize_last_body,
    grid=(G,),
    in_specs=[
        pl.BlockSpec((8, D), lambda i: (0, 0)),
        pl.BlockSpec((8, D), lambda i: (0, 0)),
        pl.BlockSpec((R, D), lambda i: (i, 0)),
        pl.BlockSpec((1, D), lambda i: (0, 0)),
        pl.BlockSpec((1, D), lambda i: (0, 0)),
    ],
    out_specs=pl.BlockSpec((R, D), lambda i: (i, 0)),
    out_shape=jax.ShapeDtypeStruct((NP, D), jnp.float32),
)


def _pool_body(bat_ref, h_ref, out_ref, sums, cnts):
    i = pl.program_id(0)

    @pl.when(i == 0)
    def _():
        sums[...] = jnp.zeros_like(sums)
        cnts[...] = jnp.zeros_like(cnts)

    brow = bat_ref[0]
    iota = lax.broadcasted_iota(jnp.int32, (B, R), 0).astype(jnp.float32)
    onehot_t = (jnp.broadcast_to(brow, (B, R)) == iota).astype(jnp.float32)
    sums[...] += lax.dot_general(onehot_t, h_ref[...], (((1,), (0,)), ((), ())),
                                 precision=lax.Precision.HIGHEST,
                                 preferred_element_type=jnp.float32)
    cnts[...] += lax.dot_general(onehot_t, jnp.ones((R, D), jnp.float32),
                                 (((1,), (0,)), ((), ())),
                                 precision=lax.Precision.HIGHEST,
                                 preferred_element_type=jnp.float32)

    @pl.when(i == G - 1)
    def _():
        out_ref[...] = sums[...] / jnp.maximum(cnts[...], 1.0)


_pool = pl.pallas_call(
    _pool_body,
    grid=(G,),
    in_specs=[
        pl.BlockSpec((1, 1, R), lambda i: (i, 0, 0)),
        pl.BlockSpec((R, D), lambda i: (i, 0)),
    ],
    out_specs=pl.BlockSpec((B, D), lambda i: (0, 0)),
    out_shape=jax.ShapeDtypeStruct((B, D), jnp.float32),
    scratch_shapes=[
        pltpu.VMEM((B, D), jnp.float32),
        pltpu.VMEM((B, D), jnp.float32),
    ],
)


def kernel(x, edge_index, edge_attr, batch, atom_embed, bond_embed,
           W1, b1, W2, b2, eps, gamma, beta):
    n_layers = W1.shape[0]
    x0f = jnp.concatenate(
        [x[:, 0].astype(jnp.float32), jnp.zeros((NP - N,), jnp.float32)]
    ).reshape(G, 1, R)
    batf = jnp.concatenate(
        [batch.astype(jnp.float32), jnp.full((NP - N,), -1.0, jnp.float32)]
    ).reshape(G, 1, R)
    src = edge_index[0].astype(jnp.int32)
    dst = edge_index[1].astype(jnp.int32)
    bnd = edge_attr[:, 0].astype(jnp.int32)

    h, t = _embed(x0f, atom_embed.astype(jnp.float32),
                  bond_embed.astype(jnp.float32))
    for l in range(n_layers):
        aggs = _edge_agg(src, bnd, dst, t.reshape(NB * NP, D))
        epsb = jnp.broadcast_to(1.0 + eps[l], (1, D))
        y, stm = _mlp(h, aggs, aggs, epsb, W1[l], b1[l].reshape(1, 2 * D),
                      W2[l], b2[l].reshape(1, D))
        stv = _var(stm, y)
        if l < n_layers - 1:
            h, t = _finalize(stm, stv, y, gamma[l].reshape(1, D),
                             beta[l].reshape(1, D),
                             bond_embed.astype(jnp.float32))
        else:
            h = _finalize_last(stm, stv, y, gamma[l].reshape(1, D),
                               beta[l].reshape(1, D))
    return _pool(batf, h)
